# baseline (speedup 1.0000x reference)
import dataclasses

import jax
import jax.numpy as jnp
from jax import lax
from jax.experimental import pallas as pl
from jax.experimental.pallas import tpu as pltpu
from jax.experimental.pallas import tpu_sc as plsc

N = 10000
E = 160000
NPAD = 10240
NG = 64
D = 256
HALF = 128
R = 256
NBLK = NPAD // R
NS = 16
K = 128
EPT = E // NS
CHUNKS = 80
EPT_PAD = CHUNKS * K
NQ = 4
NB = 2560
NBT = NQ * NB
ACC_ROWS = 2688
DUMMY_ROW = ACC_ROWS - 2
RPS = ACC_ROWS // NS
OPS = NB // NS

_mesh = plsc.VectorSubcoreMesh(core_axis_name="c", subcore_axis_name="s")

_cp_no_layout = pltpu.CompilerParams()
if "needs_layout_passes" in pltpu.CompilerParams.__dataclass_fields__:
    _cp_no_layout = dataclasses.replace(_cp_no_layout,
                                        needs_layout_passes=False)


def _part_body(src_hbm, dst_hbm, pre0_hbm, pred_hbm,
               slists_hbm, dlists_hbm, cnt_hbm, hist_hbm,
               src_v, dst_v, l0s, l0d, l1s, l1d, l2s, l2d, l3s, l3d,
               cnt_smem, hist_v):
    c = lax.axis_index("c")
    s = lax.axis_index("s")

    @pl.when(c == 0)
    def _():
        pltpu.sync_copy(src_hbm.at[s], src_v)
        pltpu.sync_copy(dst_hbm.at[s], dst_v)
        pltpu.sync_copy(pre0_hbm, l0s)
        pltpu.sync_copy(pre0_hbm, l1s)
        pltpu.sync_copy(pre0_hbm, l2s)
        pltpu.sync_copy(pre0_hbm, l3s)
        pltpu.sync_copy(pred_hbm, l0d)
        pltpu.sync_copy(pred_hbm, l1d)
        pltpu.sync_copy(pred_hbm, l2d)
        pltpu.sync_copy(pred_hbm, l3d)

        iota = lax.iota(jnp.int32, 16)
        onesf = jnp.ones((16,), jnp.float32)

        @pl.loop(0, NPAD // K, step=1)
        def _(zr):
            @pl.loop(0, K, step=16)
            def _(zc):
                hist_v[zr, pl.ds(zc, 16)] = jnp.zeros((16,), jnp.float32)

        srefs = ((l0s, l0d), (l1s, l1d), (l2s, l2d), (l3s, l3d))

        def step(i, carry):
            row = i // 8
            colo = (i % 8) * 16
            sv = src_v[row, pl.ds(colo, 16)]
            dv = dst_v[row, pl.ds(colo, 16)]
            valid = dv >= 0
            bq = ((dv >= NB).astype(jnp.int32) + (dv >= 2 * NB)
                  + (dv >= 3 * NB))
            new_carry = []
            for q in range(NQ):
                mq = jnp.logical_and(bq == q, valid) if q else \
                    jnp.logical_or(bq == 0, jnp.logical_not(valid))
                dl = jnp.where(valid, dv - q * NB, DUMMY_ROW)
                cq = plsc.cumsum(mq.astype(jnp.int32))
                idx = carry[q] + cq - 1
                r = jnp.right_shift(idx, 7)
                col = jnp.bitwise_and(idx, K - 1)
                plsc.store_scatter(srefs[q][0], [r, col], sv, mask=mq)
                plsc.store_scatter(srefs[q][1], [r, col], dl, mask=mq)
                new_carry.append(carry[q] + jnp.sum(mq.astype(jnp.int32)))
            dvc = jnp.where(valid, dv, 0)
            plsc.addupdate_scatter(hist_v,
                                   [jnp.right_shift(dvc, 7),
                                    jnp.bitwise_and(dvc, K - 1)],
                                   onesf, mask=valid)
            return tuple(new_carry)

        cnts = lax.fori_loop(0, EPT_PAD // 16, step,
                             (jnp.int32(0),) * NQ)
        for q in range(NQ):
            cnt_smem[pl.ds(16 * q, 16)] = jnp.full((16,), cnts[q], jnp.int32)
            pltpu.sync_copy(srefs[q][0], slists_hbm.at[s].at[q])
            pltpu.sync_copy(srefs[q][1], dlists_hbm.at[s].at[q])
        pltpu.sync_copy(cnt_smem, cnt_hbm.at[s])
        pltpu.sync_copy(hist_v, hist_hbm.at[s])


def _sc_part(src_slabs, dst_slabs, pre0, pred):
    return pl.kernel(
        _part_body,
        mesh=_mesh,
        compiler_params=_cp_no_layout,
        out_type=[
            jax.ShapeDtypeStruct((NS, NQ, CHUNKS, K), jnp.int32),
            jax.ShapeDtypeStruct((NS, NQ, CHUNKS, K), jnp.int32),
            jax.ShapeDtypeStruct((NS, 64), jnp.int32),
            jax.ShapeDtypeStruct((NS, NPAD // K, K), jnp.float32),
        ],
        scratch_types=[
            pltpu.VMEM((CHUNKS, K), jnp.int32),
            pltpu.VMEM((CHUNKS, K), jnp.int32),
            pltpu.VMEM((CHUNKS, K), jnp.int32),
            pltpu.VMEM((CHUNKS, K), jnp.int32),
            pltpu.VMEM((CHUNKS, K), jnp.int32),
            pltpu.VMEM((CHUNKS, K), jnp.int32),
            pltpu.VMEM((CHUNKS, K), jnp.int32),
            pltpu.VMEM((CHUNKS, K), jnp.int32),
            pltpu.VMEM((CHUNKS, K), jnp.int32),
            pltpu.VMEM((CHUNKS, K), jnp.int32),
            pltpu.VMEM((64,), jnp.int32),
            pltpu.VMEM((NPAD // K, K), jnp.float32),
        ],
    )(src_slabs, dst_slabs, pre0, pred)


def _agg_body(table_hbm, src_hbm, dst_hbm, cnt_hbm, zeros_hbm, out_hbm,
              src_v, dst_v, b0, b1, b2, b3, zeros_v, cnt_s, acc,
              gs0, gs1, gs2, gs3, ss0, ss1, ss2, ss3):
    c = lax.axis_index("c")
    s = lax.axis_index("s")
    bufs = (b0, b1, b2, b3)
    gsem = (gs0, gs1, gs2, gs3)
    ssem = (ss0, ss1, ss2, ss3)
    pltpu.sync_copy(cnt_hbm.at[s], cnt_s)
    pltpu.sync_copy(zeros_hbm, zeros_v)
    for p in range(NQ):
        pltpu.sync_copy(src_hbm.at[s].at[p], src_v)
        pltpu.sync_copy(dst_hbm.at[s].at[p], dst_v)

        @pl.when(c == 1)
        def _():
            @pl.loop(0, CHUNKS, step=1)
            def _(zr):
                @pl.loop(0, K, step=16)
                def _(zc):
                    src_v[zr, pl.ds(zc, 16)] = \
                        src_v[zr, pl.ds(zc, 16)] + NPAD

        pltpu.sync_copy(zeros_v, acc.at[pl.ds(s * RPS, K)])
        pltpu.sync_copy(zeros_v.at[pl.ds(0, RPS - K)],
                        acc.at[pl.ds(s * RPS + K, RPS - K)])
        plsc.subcore_barrier()
        cnt = jnp.max(cnt_s[pl.ds(16 * p, 16)])
        nch4 = jnp.maximum((cnt + 4 * K - 1) // (4 * K), 1)
        for i in range(4):
            pltpu.async_copy(table_hbm.at[src_v.at[i]], bufs[i], gsem[i])

        def body(t, carry):
            j = t * 4
            for i in range(4):
                pltpu.make_async_copy(table_hbm.at[pl.ds(0, K)], bufs[i],
                                      gsem[i]).wait()
                pltpu.async_copy(bufs[i], acc.at[dst_v.at[j + i]], ssem[i],
                                 add=True)
            for i in range(4):
                pltpu.make_async_copy(bufs[i], acc.at[pl.ds(0, K)],
                                      ssem[i]).wait()

                @pl.when(j + 4 + i < nch4 * 4)
                def _():
                    pltpu.async_copy(table_hbm.at[src_v.at[j + 4 + i]],
                                     bufs[i], gsem[i])
            return carry

        lax.fori_loop(0, nch4, body, 0)
        plsc.subcore_barrier()
        pltpu.sync_copy(acc.at[pl.ds(s * OPS, OPS)],
                        out_hbm.at[c].at[pl.ds(p * NB + s * OPS, OPS)])
        plsc.subcore_barrier()


def _sc_agg(table_flat, slists, dlists, cnts, zeros128):
    return pl.kernel(
        _agg_body,
        mesh=_mesh,
        compiler_params=_cp_no_layout,
        out_type=jax.ShapeDtypeStruct((2, NBT, HALF), jnp.float32),
        scratch_types=[
            pltpu.VMEM((CHUNKS, K), jnp.int32),
            pltpu.VMEM((CHUNKS, K), jnp.int32),
            pltpu.VMEM((K, HALF), jnp.float32),
            pltpu.VMEM((K, HALF), jnp.float32),
            pltpu.VMEM((K, HALF), jnp.float32),
            pltpu.VMEM((K, HALF), jnp.float32),
            pltpu.VMEM((K, HALF), jnp.float32),
            pltpu.VMEM((64,), jnp.int32),
            pltpu.VMEM_SHARED((ACC_ROWS, HALF), jnp.float32),
            pltpu.SemaphoreType.DMA,
            pltpu.SemaphoreType.DMA,
            pltpu.SemaphoreType.DMA,
            pltpu.SemaphoreType.DMA,
            pltpu.SemaphoreType.DMA,
            pltpu.SemaphoreType.DMA,
            pltpu.SemaphoreType.DMA,
            pltpu.SemaphoreType.DMA,
        ],
    )(table_flat, slists, dlists, cnts, zeros128)


def _dot_f32(a, b):
    return jnp.dot(a, b, preferred_element_type=jnp.float32)


def _dinv_of(h_ref):
    deg = lax.dot_general(h_ref[...], jnp.ones((16, 1), jnp.float32),
                          (((0,), (0,)), ((), ())),
                          preferred_element_type=jnp.float32) + 1.0
    return 1.0 / jnp.sqrt(deg)


def _mm1_body(x_ref, w_ref, h_ref, out_ref):
    dinv = _dinv_of(h_ref)
    out_ref[0] = _dot_f32(x_ref[...], w_ref[...]) * dinv


def _assemble(agg_ref, tab_ref):
    m = agg_ref[...] + tab_ref[...]
    return jnp.concatenate([m[0], m[1]], axis=1)


def _mm2_body(agg_ref, tab_ref, h_ref, b1_ref, w_ref, out_ref):
    dinv = _dinv_of(h_ref)
    a = _assemble(agg_ref, tab_ref) * dinv + b1_ref[...]
    a = jnp.maximum(a, 0.0)
    out_ref[0] = _dot_f32(a, w_ref[...]) * dinv


def _pool_body(agg_ref, tab_ref, h_ref, b2_ref, batch_ref,
               fw1_ref, fb1_ref, fw2_ref, fb2_ref, out_ref, gmax):
    i = pl.program_id(0)

    @pl.when(i == 0)
    def _():
        gmax[...] = jnp.full((NG, D), -jnp.inf, jnp.float32)

    dinv = _dinv_of(h_ref)
    h = _assemble(agg_ref, tab_ref) * dinv + b2_ref[...]
    h = jnp.maximum(h, 0.0)
    bcol = batch_ref[0]
    gmin = jnp.min(jnp.where(bcol >= 0, bcol, NG))
    gmax_id = jnp.max(bcol)

    def upd(g, carry):
        mask = bcol == g
        v = jnp.max(jnp.where(mask, h, -jnp.inf), axis=0, keepdims=True)
        gmax[pl.ds(g, 1), :] = jnp.maximum(gmax[pl.ds(g, 1), :], v)
        return carry

    lax.fori_loop(gmin, gmax_id + 1, upd, 0)

    @pl.when(i == NBLK - 1)
    def _():
        g1 = _dot_f32(gmax[...], fw1_ref[...]) + fb1_ref[...]
        g1 = jnp.maximum(g1, 0.0)
        out_ref[...] = _dot_f32(g1, fw2_ref[...]) + fb2_ref[...]


def _mm1(x_pad, W1, hist):
    return pl.pallas_call(
        _mm1_body,
        grid=(NBLK, 2),
        in_specs=[
            pl.BlockSpec((R, D), lambda i, c: (i, 0)),
            pl.BlockSpec((D, HALF), lambda i, c: (0, c)),
            pl.BlockSpec((NS, R), lambda i, c: (0, i)),
        ],
        out_specs=pl.BlockSpec((1, R, HALF), lambda i, c: (c, i, 0)),
        out_shape=jax.ShapeDtypeStruct((2, NPAD, HALF), jnp.float32),
    )(x_pad, W1, hist)


def _mm2(agg1, tab1, hist, b1r, W2):
    return pl.pallas_call(
        _mm2_body,
        grid=(NBLK, 2),
        in_specs=[
            pl.BlockSpec((2, R, HALF), lambda i, c: (0, i, 0)),
            pl.BlockSpec((2, R, HALF), lambda i, c: (0, i, 0)),
            pl.BlockSpec((NS, R), lambda i, c: (0, i)),
            pl.BlockSpec((1, D), lambda i, c: (0, 0)),
            pl.BlockSpec((D, HALF), lambda i, c: (0, c)),
        ],
        out_specs=pl.BlockSpec((1, R, HALF), lambda i, c: (c, i, 0)),
        out_shape=jax.ShapeDtypeStruct((2, NPAD, HALF), jnp.float32),
    )(agg1, tab1, hist, b1r, W2)


def _pool(agg2, tab2, hist, b2r, batch3, fcW1, fb1r, fw2p, fb2r):
    return pl.pallas_call(
        _pool_body,
        grid=(NBLK,),
        in_specs=[
            pl.BlockSpec((2, R, HALF), lambda i: (0, i, 0)),
            pl.BlockSpec((2, R, HALF), lambda i: (0, i, 0)),
            pl.BlockSpec((NS, R), lambda i: (0, i)),
            pl.BlockSpec((1, D), lambda i: (0, 0)),
            pl.BlockSpec((1, R, 1), lambda i: (i, 0, 0)),
            pl.BlockSpec((D, HALF), lambda i: (0, 0)),
            pl.BlockSpec((1, HALF), lambda i: (0, 0)),
            pl.BlockSpec((HALF, HALF), lambda i: (0, 0)),
            pl.BlockSpec((1, HALF), lambda i: (0, 0)),
        ],
        out_specs=pl.BlockSpec((NG, HALF), lambda i: (0, 0)),
        out_shape=jax.ShapeDtypeStruct((NG, HALF), jnp.float32),
        scratch_shapes=[pltpu.VMEM((NG, D), jnp.float32)],
    )(agg2, tab2, hist, b2r, batch3, fcW1, fb1r, fw2p, fb2r)


def kernel(x, edge_index, edge_attr, batch, W1, b1, W2, b2,
           fcW1, fcb1, fcW2, fcb2):
    del edge_attr
    f32 = jnp.float32

    src = edge_index[0]
    dst = edge_index[1]
    pad_e = NS * EPT_PAD - E
    src_p = jnp.concatenate([src, jnp.zeros((pad_e,), jnp.int32)])
    dst_p = jnp.concatenate([dst, jnp.full((pad_e,), -1, jnp.int32)])
    src_slabs = src_p.reshape(NS, CHUNKS, K)
    dst_slabs = dst_p.reshape(NS, CHUNKS, K)
    pre0 = jnp.zeros((CHUNKS, K), jnp.int32)
    pred = jnp.full((CHUNKS, K), DUMMY_ROW, jnp.int32)
    x_pad = jnp.pad(x, ((0, NPAD - N), (0, 0)))
    batch_p = jnp.concatenate(
        [batch, jnp.full((NPAD - N,), -1, jnp.int32)]).reshape(NBLK, R, 1)

    zeros128 = jnp.zeros((K, HALF), f32)

    b1r = b1.reshape(1, D)
    b2r = b2.reshape(1, D)
    fb1r = fcb1.reshape(1, HALF)
    fw2p = jnp.pad(fcW2, ((0, 0), (0, HALF - 1)))
    fb2r = jnp.broadcast_to(fcb2.reshape(1, 1), (1, HALF))

    slists, dlists, cnts, hist = _sc_part(src_slabs, dst_slabs, pre0, pred)
    hist = hist.reshape(NS, NPAD)

    tab1 = _mm1(x_pad, W1, hist)
    agg1 = _sc_agg(tab1.reshape(2 * NPAD, HALF), slists, dlists, cnts,
                   zeros128)[:, :NPAD, :]

    tab2 = _mm2(agg1, tab1, hist, b1r, W2)
    agg2 = _sc_agg(tab2.reshape(2 * NPAD, HALF), slists, dlists, cnts,
                   zeros128)[:, :NPAD, :]

    out_full = _pool(agg2, tab2, hist, b2r, batch_p,
                     fcW1, fb1r, fw2p, fb2r)
    return out_full[:, :1]


# speedup vs baseline: 2.1518x; 2.1518x over previous
import dataclasses

import jax
import jax.numpy as jnp
from jax import lax
from jax.experimental import pallas as pl
from jax.experimental.pallas import tpu as pltpu
from jax.experimental.pallas import tpu_sc as plsc

N = 10000
E = 160000
NPAD = 10240
NG = 64
D = 256
HALF = 128
R = 256
NBLK = NPAD // R
NS = 16
K = 128
EPT = E // NS
CHUNKS = 80
EPT_PAD = CHUNKS * K
NH = NPAD // 2
ACC_ROWS = 6144
DUMMY_ROW = ACC_ROWS - 2
ZCH = ACC_ROWS // NS // K
OPS = NH // NS

_mesh = plsc.VectorSubcoreMesh(core_axis_name="c", subcore_axis_name="s")

_cp_no_layout = pltpu.CompilerParams()
if "needs_layout_passes" in pltpu.CompilerParams.__dataclass_fields__:
    _cp_no_layout = dataclasses.replace(_cp_no_layout,
                                        needs_layout_passes=False)


def _part_body(src_hbm, dst_hbm, pre0_hbm, pre1_hbm, pred_hbm,
               slists_hbm, dlists_hbm, cnt_hbm,
               src_v, dst_v, l0s0, l0s1, l0d, l1s0, l1s1, l1d, cnt_smem):
    c = lax.axis_index("c")
    s = lax.axis_index("s")

    @pl.when(c == 0)
    def _():
        pltpu.sync_copy(src_hbm.at[s], src_v)
        pltpu.sync_copy(dst_hbm.at[s], dst_v)
        pltpu.sync_copy(pre0_hbm, l0s0)
        pltpu.sync_copy(pre0_hbm, l1s0)
        pltpu.sync_copy(pre1_hbm, l0s1)
        pltpu.sync_copy(pre1_hbm, l1s1)
        pltpu.sync_copy(pred_hbm, l0d)
        pltpu.sync_copy(pred_hbm, l1d)

        iota = lax.iota(jnp.int32, 16)

        def step(i, carry):
            cnt0, cnt1 = carry
            sv = src_v[pl.ds(i * 16, 16)]
            dv = dst_v[pl.ds(i * 16, 16)]
            valid = dv >= 0
            m0 = dv < NH
            m1 = jnp.logical_not(m0)
            dl0 = jnp.where(valid, dv, DUMMY_ROW)
            dl1 = dv - NH
            c0 = plsc.cumsum(m0.astype(jnp.int32))
            idx0 = cnt0 + c0 - 1
            idx1 = cnt1 + iota - c0
            r0 = jnp.right_shift(idx0, 7)
            q0 = jnp.bitwise_and(idx0, K - 1)
            r1 = jnp.right_shift(idx1, 7)
            q1 = jnp.bitwise_and(idx1, K - 1)
            plsc.store_scatter(l0s0, [r0, q0], sv, mask=m0)
            plsc.store_scatter(l0s1, [r0, q0], sv + NPAD, mask=m0)
            plsc.store_scatter(l0d, [r0, q0], dl0, mask=m0)
            plsc.store_scatter(l1s0, [r1, q1], sv, mask=m1)
            plsc.store_scatter(l1s1, [r1, q1], sv + NPAD, mask=m1)
            plsc.store_scatter(l1d, [r1, q1], dl1, mask=m1)
            n0 = jnp.sum(m0.astype(jnp.int32))
            return (cnt0 + n0, cnt1 + (16 - n0))

        cnt0, cnt1 = lax.fori_loop(0, EPT_PAD // 16, step,
                                   (jnp.int32(0), jnp.int32(0)))
        cnt_smem[pl.ds(0, 16)] = jnp.full((16,), cnt0, jnp.int32)
        cnt_smem[pl.ds(16, 16)] = jnp.full((16,), cnt1, jnp.int32)
        pltpu.sync_copy(l0s0, slists_hbm.at[s].at[0].at[0])
        pltpu.sync_copy(l0s1, slists_hbm.at[s].at[0].at[1])
        pltpu.sync_copy(l1s0, slists_hbm.at[s].at[1].at[0])
        pltpu.sync_copy(l1s1, slists_hbm.at[s].at[1].at[1])
        pltpu.sync_copy(l0d, dlists_hbm.at[s].at[0])
        pltpu.sync_copy(l1d, dlists_hbm.at[s].at[1])
        pltpu.sync_copy(cnt_smem, cnt_hbm.at[s])


def _sc_part(src_slabs, dst_slabs, pre0, pre1, pred):
    return pl.kernel(
        _part_body,
        mesh=_mesh,
        compiler_params=_cp_no_layout,
        out_type=[
            jax.ShapeDtypeStruct((NS, 2, 2, CHUNKS, K), jnp.int32),
            jax.ShapeDtypeStruct((NS, 2, CHUNKS, K), jnp.int32),
            jax.ShapeDtypeStruct((NS, 32), jnp.int32),
        ],
        scratch_types=[
            pltpu.VMEM((EPT_PAD,), jnp.int32),
            pltpu.VMEM((EPT_PAD,), jnp.int32),
            pltpu.VMEM((CHUNKS, K), jnp.int32),
            pltpu.VMEM((CHUNKS, K), jnp.int32),
            pltpu.VMEM((CHUNKS, K), jnp.int32),
            pltpu.VMEM((CHUNKS, K), jnp.int32),
            pltpu.VMEM((CHUNKS, K), jnp.int32),
            pltpu.VMEM((CHUNKS, K), jnp.int32),
            pltpu.VMEM((32,), jnp.int32),
        ],
    )(src_slabs, dst_slabs, pre0, pre1, pred)


def _hist_body(dst_hbm, cnt_hbm, ones_hbm, zeros_hbm, out_hbm,
               dst_v, ones_v, zeros_v, cnt_s, acc):
    c = lax.axis_index("c")
    s = lax.axis_index("s")
    pltpu.sync_copy(cnt_hbm.at[s], cnt_s)
    pltpu.sync_copy(dst_hbm.at[s].at[c], dst_v)
    pltpu.sync_copy(ones_hbm, ones_v)
    pltpu.sync_copy(zeros_hbm, zeros_v)
    for k in range(ZCH):
        pltpu.sync_copy(zeros_v,
                        acc.at[pl.ds(s * (ACC_ROWS // NS) + k * K, K)])
    plsc.subcore_barrier()
    nch = (jnp.max(cnt_s[pl.ds(16 * c, 16)]) + K - 1) // K

    def body(j, carry):
        pltpu.sync_copy(ones_v, acc.at[dst_v.at[j]], add=True)
        return carry

    lax.fori_loop(0, nch, body, 0)
    plsc.subcore_barrier()
    pltpu.sync_copy(acc.at[pl.ds(s * OPS, OPS)],
                    out_hbm.at[pl.ds(c * NH + s * OPS, OPS)])


def _agg_body(table_hbm, src_hbm, dst_hbm, cnt_hbm, zeros_hbm, out_hbm,
              src_v, dst_v, gbuf, zeros_v, cnt_s, acc):
    c = lax.axis_index("c")
    s = lax.axis_index("s")
    pltpu.sync_copy(cnt_hbm.at[s], cnt_s)
    pltpu.sync_copy(zeros_hbm, zeros_v)
    for p in range(2):
        pltpu.sync_copy(src_hbm.at[s].at[p].at[c], src_v)
        pltpu.sync_copy(dst_hbm.at[s].at[p], dst_v)
        for k in range(ZCH):
            pltpu.sync_copy(zeros_v,
                            acc.at[pl.ds(s * (ACC_ROWS // NS) + k * K, K)])
        plsc.subcore_barrier()
        nch = (jnp.max(cnt_s[pl.ds(16 * p, 16)]) + K - 1) // K

        def body(j, carry):
            pltpu.sync_copy(table_hbm.at[src_v.at[j]], gbuf)
            pltpu.sync_copy(gbuf, acc.at[dst_v.at[j]], add=True)
            return carry

        lax.fori_loop(0, nch, body, 0)
        plsc.subcore_barrier()
        pltpu.sync_copy(acc.at[pl.ds(s * OPS, OPS)],
                        out_hbm.at[c].at[pl.ds(p * NH + s * OPS, OPS)])
        plsc.subcore_barrier()


def _sc_hist(dlists, cnts, ones128, zeros128):
    return pl.kernel(
        _hist_body,
        mesh=_mesh,
        compiler_params=_cp_no_layout,
        out_type=jax.ShapeDtypeStruct((NPAD, HALF), jnp.float32),
        scratch_types=[
            pltpu.VMEM((CHUNKS, K), jnp.int32),
            pltpu.VMEM((K, HALF), jnp.float32),
            pltpu.VMEM((K, HALF), jnp.float32),
            pltpu.VMEM((32,), jnp.int32),
            pltpu.VMEM_SHARED((ACC_ROWS, HALF), jnp.float32),
        ],
    )(dlists, cnts, ones128, zeros128)


def _sc_agg(table_flat, slists, dlists, cnts, zeros128):
    return pl.kernel(
        _agg_body,
        mesh=_mesh,
        compiler_params=_cp_no_layout,
        out_type=jax.ShapeDtypeStruct((2, NPAD, HALF), jnp.float32),
        scratch_types=[
            pltpu.VMEM((CHUNKS, K), jnp.int32),
            pltpu.VMEM((CHUNKS, K), jnp.int32),
            pltpu.VMEM((K, HALF), jnp.float32),
            pltpu.VMEM((K, HALF), jnp.float32),
            pltpu.VMEM((32,), jnp.int32),
            pltpu.VMEM_SHARED((ACC_ROWS, HALF), jnp.float32),
        ],
    )(table_flat, slists, dlists, cnts, zeros128)


def _dot_f32(a, b):
    return jnp.dot(a, b, preferred_element_type=jnp.float32)


def _dinv_of(h_ref):
    deg = h_ref[:, 0:1] + 1.0
    return 1.0 / jnp.sqrt(deg)


def _mm1_body(x_ref, w_ref, h_ref, out_ref):
    dinv = _dinv_of(h_ref)
    out_ref[0] = _dot_f32(x_ref[...], w_ref[...]) * dinv


def _assemble(agg_ref, tab_ref):
    m = agg_ref[...] + tab_ref[...]
    return jnp.concatenate([m[0], m[1]], axis=1)


def _mm2_body(agg_ref, tab_ref, h_ref, b1_ref, w_ref, out_ref):
    dinv = _dinv_of(h_ref)
    a = _assemble(agg_ref, tab_ref) * dinv + b1_ref[...]
    a = jnp.maximum(a, 0.0)
    out_ref[0] = _dot_f32(a, w_ref[...]) * dinv


def _pool_body(agg_ref, tab_ref, h_ref, b2_ref, batch_ref,
               fw1_ref, fb1_ref, fw2_ref, fb2_ref, out_ref, gmax):
    i = pl.program_id(0)

    @pl.when(i == 0)
    def _():
        gmax[...] = jnp.full((NG, D), -jnp.inf, jnp.float32)

    dinv = _dinv_of(h_ref)
    h = _assemble(agg_ref, tab_ref) * dinv + b2_ref[...]
    h = jnp.maximum(h, 0.0)
    bcol = batch_ref[0]
    gmin = jnp.min(jnp.where(bcol >= 0, bcol, NG))
    gmax_id = jnp.max(bcol)

    def upd(g, carry):
        mask = bcol == g
        v = jnp.max(jnp.where(mask, h, -jnp.inf), axis=0, keepdims=True)
        gmax[pl.ds(g, 1), :] = jnp.maximum(gmax[pl.ds(g, 1), :], v)
        return carry

    lax.fori_loop(gmin, gmax_id + 1, upd, 0)

    @pl.when(i == NBLK - 1)
    def _():
        g1 = _dot_f32(gmax[...], fw1_ref[...]) + fb1_ref[...]
        g1 = jnp.maximum(g1, 0.0)
        out_ref[...] = _dot_f32(g1, fw2_ref[...]) + fb2_ref[...]


def _mm1(x_pad, W1, hist):
    return pl.pallas_call(
        _mm1_body,
        grid=(NBLK, 2),
        in_specs=[
            pl.BlockSpec((R, D), lambda i, c: (i, 0)),
            pl.BlockSpec((D, HALF), lambda i, c: (0, c)),
            pl.BlockSpec((R, HALF), lambda i, c: (i, 0)),
        ],
        out_specs=pl.BlockSpec((1, R, HALF), lambda i, c: (c, i, 0)),
        out_shape=jax.ShapeDtypeStruct((2, NPAD, HALF), jnp.float32),
    )(x_pad, W1, hist)


def _mm2(agg1, tab1, hist, b1r, W2):
    return pl.pallas_call(
        _mm2_body,
        grid=(NBLK, 2),
        in_specs=[
            pl.BlockSpec((2, R, HALF), lambda i, c: (0, i, 0)),
            pl.BlockSpec((2, R, HALF), lambda i, c: (0, i, 0)),
            pl.BlockSpec((R, HALF), lambda i, c: (i, 0)),
            pl.BlockSpec((1, D), lambda i, c: (0, 0)),
            pl.BlockSpec((D, HALF), lambda i, c: (0, c)),
        ],
        out_specs=pl.BlockSpec((1, R, HALF), lambda i, c: (c, i, 0)),
        out_shape=jax.ShapeDtypeStruct((2, NPAD, HALF), jnp.float32),
    )(agg1, tab1, hist, b1r, W2)


def _pool(agg2, tab2, hist, b2r, batch3, fcW1, fb1r, fw2p, fb2r):
    return pl.pallas_call(
        _pool_body,
        grid=(NBLK,),
        in_specs=[
            pl.BlockSpec((2, R, HALF), lambda i: (0, i, 0)),
            pl.BlockSpec((2, R, HALF), lambda i: (0, i, 0)),
            pl.BlockSpec((R, HALF), lambda i: (i, 0)),
            pl.BlockSpec((1, D), lambda i: (0, 0)),
            pl.BlockSpec((1, R, 1), lambda i: (i, 0, 0)),
            pl.BlockSpec((D, HALF), lambda i: (0, 0)),
            pl.BlockSpec((1, HALF), lambda i: (0, 0)),
            pl.BlockSpec((HALF, HALF), lambda i: (0, 0)),
            pl.BlockSpec((1, HALF), lambda i: (0, 0)),
        ],
        out_specs=pl.BlockSpec((NG, HALF), lambda i: (0, 0)),
        out_shape=jax.ShapeDtypeStruct((NG, HALF), jnp.float32),
        scratch_shapes=[pltpu.VMEM((NG, D), jnp.float32)],
    )(agg2, tab2, hist, b2r, batch3, fcW1, fb1r, fw2p, fb2r)


def kernel(x, edge_index, edge_attr, batch, W1, b1, W2, b2,
           fcW1, fcb1, fcW2, fcb2):
    del edge_attr
    f32 = jnp.float32

    src = edge_index[0]
    dst = edge_index[1]
    pad_e = NS * EPT_PAD - E
    src_p = jnp.concatenate([src, jnp.zeros((pad_e,), jnp.int32)])
    dst_p = jnp.concatenate([dst, jnp.full((pad_e,), -1, jnp.int32)])
    src_slabs = src_p.reshape(NS, EPT_PAD)
    dst_slabs = dst_p.reshape(NS, EPT_PAD)
    pre0 = jnp.zeros((CHUNKS, K), jnp.int32)
    pre1 = jnp.full((CHUNKS, K), NPAD, jnp.int32)
    pred = jnp.full((CHUNKS, K), DUMMY_ROW, jnp.int32)
    x_pad = jnp.pad(x, ((0, NPAD - N), (0, 0)))
    batch_p = jnp.concatenate(
        [batch, jnp.full((NPAD - N,), -1, jnp.int32)]).reshape(NBLK, R, 1)

    ones128 = jnp.ones((K, HALF), f32)
    zeros128 = jnp.zeros((K, HALF), f32)

    b1r = b1.reshape(1, D)
    b2r = b2.reshape(1, D)
    fb1r = fcb1.reshape(1, HALF)
    fw2p = jnp.pad(fcW2, ((0, 0), (0, HALF - 1)))
    fb2r = jnp.broadcast_to(fcb2.reshape(1, 1), (1, HALF))

    slists, dlists, cnts = _sc_part(src_slabs, dst_slabs, pre0, pre1, pred)
    hist = _sc_hist(dlists, cnts, ones128, zeros128)

    tab1 = _mm1(x_pad, W1, hist)
    agg1 = _sc_agg(tab1.reshape(2 * NPAD, HALF), slists, dlists, cnts,
                   zeros128)

    tab2 = _mm2(agg1, tab1, hist, b1r, W2)
    agg2 = _sc_agg(tab2.reshape(2 * NPAD, HALF), slists, dlists, cnts,
                   zeros128)

    out_full = _pool(agg2, tab2, hist, b2r, batch_p,
                     fcW1, fb1r, fw2p, fb2r)
    return out_full[:, :1]
